# baseline (speedup 1.0000x reference)
import functools

import jax
import jax.numpy as jnp
from jax import lax
from jax.experimental import pallas as pl
from jax.experimental.pallas import tpu as pltpu
from jax.experimental.pallas import tpu_sc as plsc

NC = 2
NS = 16
L = 16
NW = NC * NS
BATCH = 16384
F = 32
BPW = BATCH // NW
CHUNKS = BPW // L


def _mf_body(user_hbm, movie_hbm, uf_hbm, mf_hbm, out_hbm,
             uidx_v, midx_v, urows_v, mrows_v, out_v, sem_u, sem_m):
    wid = lax.axis_index("s") * NC + lax.axis_index("c")
    base = wid * BPW

    pltpu.sync_copy(user_hbm.at[pl.ds(base, BPW)], uidx_v)
    pltpu.sync_copy(movie_hbm.at[pl.ds(base, BPW)], midx_v)
    cu = pltpu.async_copy(uf_hbm.at[uidx_v], urows_v, sem_u)
    cm = pltpu.async_copy(mf_hbm.at[midx_v], mrows_v, sem_m)
    cu.wait()
    cm.wait()

    iota = lax.iota(jnp.int32, L)

    def chunk_body(c, carry):
        row_ids = c * L + iota
        acc = jnp.zeros((L,), jnp.float32)
        for f in range(F):
            col = jnp.full((L,), f, jnp.int32)
            uv = plsc.load_gather(urows_v, [row_ids, col])
            mv = plsc.load_gather(mrows_v, [row_ids, col])
            acc = acc + uv * mv
        out_v[pl.ds(c * L, L)] = acc
        return carry

    lax.fori_loop(0, CHUNKS, chunk_body, 0)
    pltpu.sync_copy(out_v, out_hbm.at[pl.ds(base, BPW)])


@jax.jit
def kernel(user, movie, user_factors, movie_factors):
    mesh = plsc.VectorSubcoreMesh(
        core_axis_name="c", subcore_axis_name="s",
        num_cores=NC, num_subcores=NS)
    run = pl.kernel(
        _mf_body,
        out_type=jax.ShapeDtypeStruct((BATCH,), jnp.float32),
        mesh=mesh,
        scratch_types=[
            pltpu.VMEM((BPW,), jnp.int32),
            pltpu.VMEM((BPW,), jnp.int32),
            pltpu.VMEM((BPW, F), jnp.float32),
            pltpu.VMEM((BPW, F), jnp.float32),
            pltpu.VMEM((BPW,), jnp.float32),
            pltpu.SemaphoreType.DMA,
            pltpu.SemaphoreType.DMA,
        ],
        compiler_params=pltpu.CompilerParams(
            needs_layout_passes=False, use_tc_tiling_on_sc=False),
    )
    return run(user, movie, user_factors, movie_factors)
